# Initial kernel scaffold; baseline (speedup 1.0000x reference)
import jax
import jax.numpy as jnp
from jax.experimental import pallas as pl


def kernel(x):
    raise NotImplementedError("write your pallas kernel here")


# speedup vs baseline: 2.4392x; 2.4392x over previous
import jax
import jax.numpy as jnp
from jax.experimental import pallas as pl

_INTERVAL = 4
_BLOCK_ROWS = 512


def _mask_kernel(x_ref, o_ref):
    rows = jax.lax.broadcasted_iota(jnp.int32, x_ref.shape, 0)
    keep = (rows % _INTERVAL) != 0
    o_ref[...] = jnp.where(keep, x_ref[...], 0.0)


def kernel(x):
    n, d = x.shape
    return pl.pallas_call(
        _mask_kernel,
        grid=(n // _BLOCK_ROWS,),
        in_specs=[pl.BlockSpec((_BLOCK_ROWS, d), lambda i: (i, 0))],
        out_specs=pl.BlockSpec((_BLOCK_ROWS, d), lambda i: (i, 0)),
        out_shape=jax.ShapeDtypeStruct((n, d), x.dtype),
    )(x)


# speedup vs baseline: 2.4882x; 1.0201x over previous
import jax
import jax.numpy as jnp
from jax.experimental import pallas as pl

_INTERVAL = 4
_BLOCK_ROWS = 1024


def _mask_kernel(x_ref, o_ref):
    rows = jax.lax.broadcasted_iota(jnp.int32, x_ref.shape, 0)
    keep = (rows % _INTERVAL) != 0
    o_ref[...] = jnp.where(keep, x_ref[...], 0.0)


def kernel(x):
    n, d = x.shape
    return pl.pallas_call(
        _mask_kernel,
        grid=(n // _BLOCK_ROWS,),
        in_specs=[pl.BlockSpec((_BLOCK_ROWS, d), lambda i: (i, 0))],
        out_specs=pl.BlockSpec((_BLOCK_ROWS, d), lambda i: (i, 0)),
        out_shape=jax.ShapeDtypeStruct((n, d), x.dtype),
    )(x)
